# Initial kernel scaffold; baseline (speedup 1.0000x reference)
import functools

import jax
import jax.numpy as jnp
from jax import lax
from jax.experimental import pallas as pl
from jax.experimental.pallas import tpu as pltpu
from jax.experimental.pallas import tpu_sc as plsc

VOCAB = 4608
HIDDEN = 384
EPS = 1e-6

NUM_CORES = 2
NUM_SUBCORES = 16
NUM_WORKERS = NUM_CORES * NUM_SUBCORES


def _normalize_table_body(table_ref, w_ref, out_ref):
    x = table_ref[...]
    var = jnp.mean(x * x, axis=-1, keepdims=True)
    out_ref[...] = x * lax.rsqrt(var + EPS) * w_ref[...][None, :]


def _normalize_table(table, w):
    return pl.pallas_call(
        _normalize_table_body,
        out_shape=jax.ShapeDtypeStruct((VOCAB, HIDDEN), jnp.float32),
    )(table, w)


def _make_gather(B, chunk):
    b_per_w = B // NUM_WORKERS
    n_chunks = b_per_w // chunk
    mesh = plsc.VectorSubcoreMesh(core_axis_name="c", subcore_axis_name="s")

    @functools.partial(
        pl.kernel,
        mesh=mesh,
        out_type=jax.ShapeDtypeStruct((B, HIDDEN), jnp.float32),
        scratch_types=[
            pltpu.VMEM((b_per_w,), jnp.int32),
            pltpu.VMEM((chunk, HIDDEN), jnp.float32),
            pltpu.VMEM((chunk, HIDDEN), jnp.float32),
            pltpu.SemaphoreType.DMA,
            pltpu.SemaphoreType.DMA,
        ],
    )
    def gather_kernel(idx_hbm, table_hbm, out_hbm, idx_v, buf0, buf1, g_sem, s_sem):
        wid = lax.axis_index("s") * NUM_CORES + lax.axis_index("c")
        base = wid * b_per_w
        pltpu.sync_copy(idx_hbm.at[pl.ds(base, b_per_w)], idx_v)
        bufs = (buf0, buf1)
        pltpu.async_copy(table_hbm.at[idx_v.at[pl.ds(0, chunk)]], buf0, g_sem)
        for c in range(n_chunks):
            buf = bufs[c % 2]
            nxt = bufs[(c + 1) % 2]
            pltpu.make_async_copy(table_hbm.at[idx_v.at[pl.ds(c * chunk, chunk)]],
                                  buf, g_sem).wait()
            if c + 1 < n_chunks:
                if c >= 1:
                    pltpu.make_async_copy(
                        nxt, out_hbm.at[pl.ds(base + (c - 1) * chunk, chunk)],
                        s_sem).wait()
                pltpu.async_copy(
                    table_hbm.at[idx_v.at[pl.ds((c + 1) * chunk, chunk)]],
                    nxt, g_sem)
            pltpu.async_copy(buf, out_hbm.at[pl.ds(base + c * chunk, chunk)],
                             s_sem)
        for c in range(max(0, n_chunks - 2), n_chunks):
            pltpu.make_async_copy(
                bufs[c % 2], out_hbm.at[pl.ds(base + c * chunk, chunk)],
                s_sem).wait()

    return gather_kernel


def kernel(input_ids, tok_embeddings, norm_weight):
    B, S = input_ids.shape
    flat_ids = input_ids.reshape(B * S).astype(jnp.int32)
    norm_table = _normalize_table(tok_embeddings, norm_weight)
    gather = _make_gather(B * S, chunk=128)
    out = gather(flat_ids, norm_table)
    return out.reshape(B, S, HIDDEN)


# speedup vs baseline: 2.8631x; 2.8631x over previous
import functools

import jax
import jax.numpy as jnp
from jax import lax
from jax.experimental import pallas as pl
from jax.experimental.pallas import tpu as pltpu
from jax.experimental.pallas import tpu_sc as plsc

VOCAB = 4608
HIDDEN = 384
EPS = 1e-6

NUM_CORES = 2
NUM_SUBCORES = 16
NUM_WORKERS = NUM_CORES * NUM_SUBCORES


def _normalize_table_body(table_ref, w_ref, out_ref):
    x = table_ref[...]
    var = jnp.mean(x * x, axis=-1, keepdims=True)
    out_ref[...] = x * lax.rsqrt(var + EPS) * w_ref[...][None, :]


def _normalize_table(table, w):
    return pl.pallas_call(
        _normalize_table_body,
        out_shape=jax.ShapeDtypeStruct((VOCAB, HIDDEN), jnp.float32),
    )(table, w)


def _make_gather(B, chunk):
    b_per_w = B // NUM_WORKERS
    n_chunks = b_per_w // chunk
    mesh = plsc.VectorSubcoreMesh(
        core_axis_name="c", subcore_axis_name="s",
        num_cores=NUM_CORES, num_subcores=NUM_SUBCORES)

    @functools.partial(
        pl.kernel,
        mesh=mesh,
        out_type=jax.ShapeDtypeStruct((B, HIDDEN), jnp.float32),
        scratch_types=[
            pltpu.VMEM((b_per_w,), jnp.int32),
            pltpu.VMEM((chunk, HIDDEN), jnp.float32),
            pltpu.VMEM((chunk, HIDDEN), jnp.float32),
            pltpu.SemaphoreType.DMA,
            pltpu.SemaphoreType.DMA,
        ],
    )
    def gather_kernel(idx_hbm, table_hbm, out_hbm, idx_v, buf0, buf1, g_sem, s_sem):
        wid = lax.axis_index("s") * NUM_CORES + lax.axis_index("c")
        base = wid * b_per_w
        pltpu.sync_copy(idx_hbm.at[pl.ds(base, b_per_w)], idx_v)
        bufs = (buf0, buf1)
        pltpu.async_copy(table_hbm.at[idx_v.at[pl.ds(0, chunk)]], buf0, g_sem)
        for c in range(n_chunks):
            buf = bufs[c % 2]
            nxt = bufs[(c + 1) % 2]
            pltpu.make_async_copy(table_hbm.at[idx_v.at[pl.ds(c * chunk, chunk)]],
                                  buf, g_sem).wait()
            if c + 1 < n_chunks:
                if c >= 1:
                    pltpu.make_async_copy(
                        nxt, out_hbm.at[pl.ds(base + (c - 1) * chunk, chunk)],
                        s_sem).wait()
                pltpu.async_copy(
                    table_hbm.at[idx_v.at[pl.ds((c + 1) * chunk, chunk)]],
                    nxt, g_sem)
            pltpu.async_copy(buf, out_hbm.at[pl.ds(base + c * chunk, chunk)],
                             s_sem)
        for c in range(max(0, n_chunks - 2), n_chunks):
            pltpu.make_async_copy(
                bufs[c % 2], out_hbm.at[pl.ds(base + c * chunk, chunk)],
                s_sem).wait()

    return gather_kernel


def kernel(input_ids, tok_embeddings, norm_weight):
    B, S = input_ids.shape
    flat_ids = input_ids.reshape(B * S).astype(jnp.int32)
    norm_table = _normalize_table(tok_embeddings, norm_weight)
    gather = _make_gather(B * S, chunk=128)
    out = gather(flat_ids, norm_table)
    return out.reshape(B, S, HIDDEN)


# speedup vs baseline: 3.0027x; 1.0488x over previous
import functools

import jax
import jax.numpy as jnp
from jax import lax
from jax.experimental import pallas as pl
from jax.experimental.pallas import tpu as pltpu
from jax.experimental.pallas import tpu_sc as plsc

VOCAB = 4608
HIDDEN = 384
EPS = 1e-6

NUM_CORES = 2
NUM_SUBCORES = 16
NUM_WORKERS = NUM_CORES * NUM_SUBCORES


def _normalize_table_body(table_ref, w_ref, out_ref):
    x = table_ref[...]
    var = jnp.mean(x * x, axis=-1, keepdims=True)
    out_ref[...] = x * lax.rsqrt(var + EPS) * w_ref[...][None, :]


def _normalize_table(table, w):
    return pl.pallas_call(
        _normalize_table_body,
        out_shape=jax.ShapeDtypeStruct((VOCAB, HIDDEN), jnp.float32),
    )(table, w)


def _make_gather(B, chunk, nbuf):
    b_per_w = B // NUM_WORKERS
    n_chunks = b_per_w // chunk
    mesh = plsc.VectorSubcoreMesh(
        core_axis_name="c", subcore_axis_name="s",
        num_cores=NUM_CORES, num_subcores=NUM_SUBCORES)

    @functools.partial(
        pl.kernel,
        mesh=mesh,
        out_type=jax.ShapeDtypeStruct((B, HIDDEN), jnp.float32),
        scratch_types=[
            pltpu.VMEM((b_per_w,), jnp.int32),
        ] + [pltpu.VMEM((chunk, HIDDEN), jnp.float32) for _ in range(nbuf)] + [
            pltpu.SemaphoreType.DMA,
            pltpu.SemaphoreType.DMA,
        ],
    )
    def gather_kernel(idx_hbm, table_hbm, out_hbm, idx_v, *rest):
        bufs = rest[:nbuf]
        g_sem, s_sem = rest[nbuf], rest[nbuf + 1]
        wid = lax.axis_index("s") * NUM_CORES + lax.axis_index("c")
        base = wid * b_per_w
        pltpu.sync_copy(idx_hbm.at[pl.ds(base, b_per_w)], idx_v)

        def gather_start(c):
            pltpu.async_copy(
                table_hbm.at[idx_v.at[pl.ds(c * chunk, chunk)]],
                bufs[c % nbuf], g_sem)

        def gather_wait(c):
            pltpu.make_async_copy(
                table_hbm.at[idx_v.at[pl.ds(c * chunk, chunk)]],
                bufs[c % nbuf], g_sem).wait()

        def out_start(c):
            pltpu.async_copy(
                bufs[c % nbuf], out_hbm.at[pl.ds(base + c * chunk, chunk)],
                s_sem)

        def out_wait(c):
            pltpu.make_async_copy(
                bufs[c % nbuf], out_hbm.at[pl.ds(base + c * chunk, chunk)],
                s_sem).wait()

        for c in range(min(nbuf, n_chunks)):
            gather_start(c)
        for c in range(n_chunks):
            gather_wait(c)
            out_start(c)
            nxt = c + nbuf
            if nxt < n_chunks:
                out_wait(nxt - nbuf)
                gather_start(nxt)
        for c in range(max(0, n_chunks - nbuf), n_chunks):
            out_wait(c)

    return gather_kernel


def kernel(input_ids, tok_embeddings, norm_weight):
    B, S = input_ids.shape
    flat_ids = input_ids.reshape(B * S).astype(jnp.int32)
    norm_table = _normalize_table(tok_embeddings, norm_weight)
    gather = _make_gather(B * S, chunk=64, nbuf=5)
    out = gather(flat_ids, norm_table)
    return out.reshape(B, S, HIDDEN)
